# baseline (speedup 1.0000x reference)
import functools

import jax
import jax.numpy as jnp
from jax import lax
from jax.experimental import pallas as pl
from jax.experimental.pallas import tpu as pltpu
from jax.experimental.pallas import tpu_sc as plsc

N = 10000
D = 128
HID = 128
NCLS = 10
PERM = 16
NPAD = 10240
E = 320000
NC = 2
NS = 16
L = 16
B = 128
NSLOT = 2
EPW = 10240
EPAD = NC * NS * EPW
NB = EPW // B
EPW0 = 15360
EPW1 = EPW * NC - EPW0
RPT = NPAD // NS
BR = 256


def _worker_slices(cid, sid):
    return sid * RPT, (cid * NS + sid) * EPW


def _gather_split(cid, sid):
    ebase = jnp.where(cid == 0, sid * EPW0, NS * EPW0 + sid * EPW1)
    nb = jnp.where(cid == 0, EPW0 // B, EPW1 // B)
    return ebase, nb


def _pipelined_edges(ebase, nb, tab_hbm, gidx_hbm, sidx_hbm, acc_s,
                     gi, si, rows, gsem, isg, iss):
    def _i_start(b, g):
        b0 = ebase + g * B
        pltpu.async_copy(gidx_hbm.at[pl.ds(b0, B)], gi[b], isg[b])
        pltpu.async_copy(sidx_hbm.at[pl.ds(b0, B)], si[b], iss[b])

    def _i_wait(b):
        pltpu.make_async_copy(gidx_hbm.at[pl.ds(0, B)], gi[b], isg[b]).wait()
        pltpu.make_async_copy(sidx_hbm.at[pl.ds(0, B)], si[b], iss[b]).wait()

    def _g_start(b):
        pltpu.async_copy(tab_hbm.at[gi[b]], rows[b], gsem[b])

    def _g_wait(b):
        pltpu.make_async_copy(tab_hbm.at[gi[b]], rows[b], gsem[b]).wait()

    for r in range(NSLOT - 1):
        _i_start(r, r)
        _i_wait(r)
        _g_start(r)
    _i_start(NSLOT - 1, NSLOT - 1)

    def _body(b, g):
        bn = (b + NSLOT - 1) % NSLOT
        _g_wait(b)
        _i_wait(bn)
        _g_start(bn)
        pltpu.sync_copy(rows[b], acc_s.at[si[b]], add=True)
        _i_start(b, g + NSLOT)

    def _outer(go, carry):
        for u in range(NSLOT):
            _body(u, NSLOT * go + u)
        return carry
    lax.fori_loop(0, nb // NSLOT, _outer, 0)
    for r in range(NSLOT - 1):
        _g_wait(r)
    _i_wait(NSLOT - 1)


def _edge_agg(h_hbm, src_hbm, dst_hbm, z2_hbm, agg_out, agg_s, *bufs):
    cid = lax.axis_index("c")
    sid = lax.axis_index("s")
    r0, _ = _worker_slices(cid, sid)
    ebase, nb = _gather_split(cid, sid)
    pltpu.sync_copy(z2_hbm.at[pl.ds(r0, RPT)], agg_s.at[pl.ds(r0, RPT)])
    plsc.subcore_barrier()
    gi = bufs[0:NSLOT]
    si = bufs[NSLOT:2 * NSLOT]
    rows = bufs[2 * NSLOT:3 * NSLOT]
    gsem = bufs[3 * NSLOT:4 * NSLOT]
    isg = bufs[4 * NSLOT:5 * NSLOT]
    iss = bufs[5 * NSLOT:6 * NSLOT]
    _pipelined_edges(ebase, nb, h_hbm, src_hbm, dst_hbm, agg_s,
                     gi, si, rows, gsem, isg, iss)
    plsc.subcore_barrier()
    pltpu.sync_copy(agg_s.at[pl.ds(r0, RPT)], agg_out.at[cid, pl.ds(r0, RPT)])


def _edge_deg(dst_hbm, zn_hbm,
              deg_out,
              deg_s, di0, di1, ones, isd0, isd1):
    cid = lax.axis_index("c")
    sid = lax.axis_index("s")
    r0, ebase = _worker_slices(cid, sid)
    pltpu.sync_copy(zn_hbm.at[pl.ds(r0, RPT)], deg_s.at[pl.ds(r0, RPT)])

    def _init_ones(i, carry):
        def _lane(k, inner):
            ones[i, pl.ds(k * L, L)] = jnp.ones((L,), jnp.float32)
            return inner
        return lax.fori_loop(0, D // L, _lane, carry)
    lax.fori_loop(0, B, _init_ones, 0)
    plsc.subcore_barrier()

    di = (di0, di1)
    isd = (isd0, isd1)

    def _i_start(b, g):
        pltpu.async_copy(dst_hbm.at[pl.ds(ebase + g * B, B)], di[b], isd[b])

    def _i_wait(b):
        pltpu.make_async_copy(dst_hbm.at[pl.ds(0, B)], di[b], isd[b]).wait()

    _i_start(0, 0)

    def _body(b, g):
        _i_wait(b)
        _i_start(1 - b, g + 1)
        pltpu.sync_copy(ones, deg_s.at[di[b]], add=True)

    def _outer(go, carry):
        _body(0, 2 * go)
        _body(1, 2 * go + 1)
        return carry
    lax.fori_loop(0, NB // 2, _outer, 0)
    _i_wait(0)
    plsc.subcore_barrier()
    pltpu.sync_copy(deg_s.at[pl.ds(r0, RPT)], deg_out.at[cid, pl.ds(r0, RPT)])


def _edge_coef(src_hbm, dst_hbm, degp_hbm, zn_hbm,
               c_out, r_out, c_s, *bufs):
    cid = lax.axis_index("c")
    sid = lax.axis_index("s")
    r0, _ = _worker_slices(cid, sid)
    ebase, nb = _gather_split(cid, sid)
    gi = bufs[0:NSLOT]
    si = bufs[NSLOT:2 * NSLOT]
    rows = bufs[2 * NSLOT:3 * NSLOT]
    gsem = bufs[3 * NSLOT:4 * NSLOT]
    isg = bufs[4 * NSLOT:5 * NSLOT]
    iss = bufs[5 * NSLOT:6 * NSLOT]
    d0, rbuf = rows[0], rows[1]
    pltpu.sync_copy(zn_hbm.at[pl.ds(r0, RPT)], c_s.at[pl.ds(r0, RPT)])
    def _chunk(t, carry):
        q0 = r0 + t * B
        pltpu.sync_copy(degp_hbm.at[0, pl.ds(q0, B)], d0)
        pltpu.sync_copy(degp_hbm.at[1, pl.ds(q0, B)], rbuf)

        def _rcomp(i, inner):
            def _lane(k, inner2):
                s = pl.ds(k * L, L)
                rbuf[i, s] = 1.0 / jnp.maximum(d0[i, s] + rbuf[i, s], 1.0)
                return inner2
            return lax.fori_loop(0, D // L, _lane, inner)
        lax.fori_loop(0, B, _rcomp, 0)
        pltpu.sync_copy(rbuf, r_out.at[cid, pl.ds(q0, B)])
        return carry
    lax.fori_loop(0, RPT // B, _chunk, 0)
    plsc.subcore_barrier()
    _pipelined_edges(ebase, nb, r_out.at[cid], dst_hbm, src_hbm, c_s,
                     gi, si, rows, gsem, isg, iss)
    plsc.subcore_barrier()
    pltpu.sync_copy(c_s.at[pl.ds(r0, RPT)], c_out.at[cid, pl.ds(r0, RPT)])


@functools.cache
def _sc_calls():
    mesh = plsc.VectorSubcoreMesh(
        core_axis_name="c", subcore_axis_name="s",
        num_cores=NC, num_subcores=NS)
    idx2 = [pltpu.VMEM((B,), jnp.int32)] * (2 * NSLOT)
    rows2 = [pltpu.VMEM((B, D), jnp.float32)] * NSLOT
    sems6 = [pltpu.SemaphoreType.DMA] * (3 * NSLOT)
    edge_agg = pl.kernel(
        _edge_agg,
        out_type=jax.ShapeDtypeStruct((NC, NPAD, D), jnp.float32),
        mesh=mesh,
        scratch_types=(
            [pltpu.VMEM_SHARED((NPAD, D), jnp.float32)]
            + idx2 + rows2 + sems6),
    )
    edge_deg = pl.kernel(
        _edge_deg,
        out_type=jax.ShapeDtypeStruct((NC, NPAD, D), jnp.float32),
        mesh=mesh,
        scratch_types=[
            pltpu.VMEM_SHARED((NPAD, D), jnp.float32),
            pltpu.VMEM((B,), jnp.int32),
            pltpu.VMEM((B,), jnp.int32),
            pltpu.VMEM((B, D), jnp.float32),
            pltpu.SemaphoreType.DMA,
            pltpu.SemaphoreType.DMA,
        ],
    )
    edge_coef = pl.kernel(
        _edge_coef,
        out_type=(jax.ShapeDtypeStruct((NC, NPAD, D), jnp.float32),
                  jax.ShapeDtypeStruct((NC, NPAD, D), jnp.float32)),
        mesh=mesh,
        scratch_types=(
            [pltpu.VMEM_SHARED((NPAD, D), jnp.float32)]
            + idx2 + rows2 + sems6),
    )
    return edge_agg, edge_deg, edge_coef


def _dense_body(h_ref, agg_ref, r_ref, c_ref,
                w1s_ref, w1n_ref, b1_ref, w2s_ref, w2n_ref, b2_ref,
                wc_ref, pf_ref, bc_ref,
                out_ref, s0_ref, s1_ref):
    i = pl.program_id(0)

    @pl.when(i == 0)
    def _():
        s0_ref[...] = jnp.zeros_like(s0_ref)
        s1_ref[...] = jnp.zeros_like(s1_ref)

    agg = agg_ref[0] + agg_ref[1]
    hn = agg * r_ref[:, :1]
    h1 = h_ref[...] @ w1s_ref[...] + hn @ w1n_ref[...] + b1_ref[...]
    h1 = jnp.maximum(h1, 0.0)
    row = i * BR + lax.broadcasted_iota(jnp.int32, (BR, 1), 0)
    valid = (row < N).astype(jnp.float32)
    s0_ref[...] += jnp.sum(h1 * valid, axis=0, keepdims=True)
    cc = (c_ref[0, :, :1] + c_ref[1, :, :1]) * valid
    s1_ref[...] += jnp.sum(h1 * cc, axis=0, keepdims=True)

    @pl.when(i == NPAD // BR - 1)
    def _():
        inv_n = 1.0 / N
        hg = s0_ref[...] * inv_n
        m1 = s1_ref[...] * inv_n
        h2m = hg @ w2s_ref[...] + m1 @ w2n_ref[...] + b2_ref[...]
        out_ref[...] = (h2m @ wc_ref[:D, :] + pf_ref[...] @ wc_ref[D:, :]
                        + bc_ref[...])


_dense_call = pl.pallas_call(
    _dense_body,
    grid=(NPAD // BR,),
    in_specs=[
        pl.BlockSpec((BR, D), lambda i: (i, 0)),
        pl.BlockSpec((NC, BR, D), lambda i: (0, i, 0)),
        pl.BlockSpec((BR, D), lambda i: (i, 0)),
        pl.BlockSpec((NC, BR, D), lambda i: (0, i, 0)),
        pl.BlockSpec((D, HID), lambda i: (0, 0)),
        pl.BlockSpec((D, HID), lambda i: (0, 0)),
        pl.BlockSpec((1, HID), lambda i: (0, 0)),
        pl.BlockSpec((HID, HID), lambda i: (0, 0)),
        pl.BlockSpec((HID, HID), lambda i: (0, 0)),
        pl.BlockSpec((1, HID), lambda i: (0, 0)),
        pl.BlockSpec((HID + PERM, NCLS), lambda i: (0, 0)),
        pl.BlockSpec((1, PERM), lambda i: (0, 0)),
        pl.BlockSpec((1, NCLS), lambda i: (0, 0)),
    ],
    out_specs=pl.BlockSpec((1, NCLS), lambda i: (0, 0)),
    out_shape=jax.ShapeDtypeStruct((1, NCLS), jnp.float32),
    scratch_shapes=[
        pltpu.VMEM((1, HID), jnp.float32),
        pltpu.VMEM((1, HID), jnp.float32),
    ],
)


def kernel(h, edge_index, perm_features, W1_self, W1_neigh, b1,
           W2_self, W2_neigh, b2, Wc, bc):
    edge_agg, edge_deg, edge_coef = _sc_calls()
    ei = edge_index.astype(jnp.int32)
    pad_idx = jnp.full((EPAD + NSLOT * B - E,), N, jnp.int32)
    src = jnp.concatenate([ei[0], pad_idx])
    dst = jnp.concatenate([ei[1], pad_idx])
    h_pad = jnp.pad(h, ((0, NPAD - N), (0, 0)))
    z2 = jnp.zeros((NPAD, D), jnp.float32)

    agg_parts = edge_agg(h_pad, src, dst, z2)
    deg_parts = edge_deg(dst, z2)
    c_parts, r_tab = edge_coef(src, dst, deg_parts, z2)

    return _dense_call(
        h_pad, agg_parts, r_tab[0], c_parts,
        W1_self, W1_neigh, b1.reshape(1, HID),
        W2_self, W2_neigh, b2.reshape(1, HID),
        Wc, perm_features, bc.reshape(1, NCLS))


# speedup vs baseline: 1.0362x; 1.0362x over previous
import functools

import jax
import jax.numpy as jnp
from jax import lax
from jax.experimental import pallas as pl
from jax.experimental.pallas import tpu as pltpu
from jax.experimental.pallas import tpu_sc as plsc

N = 10000
D = 128
HID = 128
NCLS = 10
PERM = 16
NPAD = 10240
E = 320000
NC = 2
NS = 16
L = 16
B = 128
NSLOT = 2
EPW = 10240
EPAD = NC * NS * EPW
NB = EPW // B
EPW0 = 17408
EPW1 = EPW * NC - EPW0
RPT = NPAD // NS
BR = 256


def _worker_slices(cid, sid):
    return sid * RPT, (cid * NS + sid) * EPW


def _gather_split(cid, sid):
    ebase = jnp.where(cid == 0, sid * EPW0, NS * EPW0 + sid * EPW1)
    nb = jnp.where(cid == 0, EPW0 // B, EPW1 // B)
    return ebase, nb


def _pipelined_edges(ebase, nb, tab_hbm, gidx_hbm, sidx_hbm, acc_s,
                     gi, si, rows, gsem, isg, iss):
    def _i_start(b, g):
        b0 = ebase + g * B
        pltpu.async_copy(gidx_hbm.at[pl.ds(b0, B)], gi[b], isg[b])
        pltpu.async_copy(sidx_hbm.at[pl.ds(b0, B)], si[b], iss[b])

    def _i_wait(b):
        pltpu.make_async_copy(gidx_hbm.at[pl.ds(0, B)], gi[b], isg[b]).wait()
        pltpu.make_async_copy(sidx_hbm.at[pl.ds(0, B)], si[b], iss[b]).wait()

    def _g_start(b):
        pltpu.async_copy(tab_hbm.at[gi[b]], rows[b], gsem[b])

    def _g_wait(b):
        pltpu.make_async_copy(tab_hbm.at[gi[b]], rows[b], gsem[b]).wait()

    for r in range(NSLOT - 1):
        _i_start(r, r)
        _i_wait(r)
        _g_start(r)
    _i_start(NSLOT - 1, NSLOT - 1)

    def _body(b, g):
        bn = (b + NSLOT - 1) % NSLOT
        _g_wait(b)
        _i_wait(bn)
        _g_start(bn)
        pltpu.sync_copy(rows[b], acc_s.at[si[b]], add=True)
        _i_start(b, g + NSLOT)

    def _outer(go, carry):
        for u in range(NSLOT):
            _body(u, NSLOT * go + u)
        return carry
    lax.fori_loop(0, nb // NSLOT, _outer, 0)
    for r in range(NSLOT - 1):
        _g_wait(r)
    _i_wait(NSLOT - 1)


def _edge_agg(h_hbm, src_hbm, dst_hbm, z2_hbm, agg_out, agg_s, *bufs):
    cid = lax.axis_index("c")
    sid = lax.axis_index("s")
    r0, _ = _worker_slices(cid, sid)
    ebase, nb = _gather_split(cid, sid)
    pltpu.sync_copy(z2_hbm.at[pl.ds(r0, RPT)], agg_s.at[pl.ds(r0, RPT)])
    plsc.subcore_barrier()
    gi = bufs[0:NSLOT]
    si = bufs[NSLOT:2 * NSLOT]
    rows = bufs[2 * NSLOT:3 * NSLOT]
    gsem = bufs[3 * NSLOT:4 * NSLOT]
    isg = bufs[4 * NSLOT:5 * NSLOT]
    iss = bufs[5 * NSLOT:6 * NSLOT]
    _pipelined_edges(ebase, nb, h_hbm, src_hbm, dst_hbm, agg_s,
                     gi, si, rows, gsem, isg, iss)
    plsc.subcore_barrier()
    pltpu.sync_copy(agg_s.at[pl.ds(r0, RPT)], agg_out.at[cid, pl.ds(r0, RPT)])


def _edge_deg(dst_hbm, zn_hbm,
              deg_out,
              deg_s, di0, di1, ones, isd0, isd1):
    cid = lax.axis_index("c")
    sid = lax.axis_index("s")
    r0, ebase = _worker_slices(cid, sid)
    pltpu.sync_copy(zn_hbm.at[pl.ds(r0, RPT)], deg_s.at[pl.ds(r0, RPT)])

    def _init_ones(i, carry):
        def _lane(k, inner):
            ones[i, pl.ds(k * L, L)] = jnp.ones((L,), jnp.float32)
            return inner
        return lax.fori_loop(0, D // L, _lane, carry)
    lax.fori_loop(0, B, _init_ones, 0)
    plsc.subcore_barrier()

    di = (di0, di1)
    isd = (isd0, isd1)

    def _i_start(b, g):
        pltpu.async_copy(dst_hbm.at[pl.ds(ebase + g * B, B)], di[b], isd[b])

    def _i_wait(b):
        pltpu.make_async_copy(dst_hbm.at[pl.ds(0, B)], di[b], isd[b]).wait()

    _i_start(0, 0)

    def _body(b, g):
        _i_wait(b)
        _i_start(1 - b, g + 1)
        pltpu.sync_copy(ones, deg_s.at[di[b]], add=True)

    def _outer(go, carry):
        _body(0, 2 * go)
        _body(1, 2 * go + 1)
        return carry
    lax.fori_loop(0, NB // 2, _outer, 0)
    _i_wait(0)
    plsc.subcore_barrier()
    pltpu.sync_copy(deg_s.at[pl.ds(r0, RPT)], deg_out.at[cid, pl.ds(r0, RPT)])


def _edge_coef(src_hbm, dst_hbm, degp_hbm, zn_hbm,
               c_out, r_out, c_s, *bufs):
    cid = lax.axis_index("c")
    sid = lax.axis_index("s")
    r0, _ = _worker_slices(cid, sid)
    ebase, nb = _gather_split(cid, sid)
    gi = bufs[0:NSLOT]
    si = bufs[NSLOT:2 * NSLOT]
    rows = bufs[2 * NSLOT:3 * NSLOT]
    gsem = bufs[3 * NSLOT:4 * NSLOT]
    isg = bufs[4 * NSLOT:5 * NSLOT]
    iss = bufs[5 * NSLOT:6 * NSLOT]
    d0, rbuf = rows[0], rows[1]
    pltpu.sync_copy(zn_hbm.at[pl.ds(r0, RPT)], c_s.at[pl.ds(r0, RPT)])
    def _chunk(t, carry):
        q0 = r0 + t * B
        pltpu.sync_copy(degp_hbm.at[0, pl.ds(q0, B)], d0)
        pltpu.sync_copy(degp_hbm.at[1, pl.ds(q0, B)], rbuf)

        def _rcomp(i, inner):
            def _lane(k, inner2):
                s = pl.ds(k * L, L)
                rbuf[i, s] = 1.0 / jnp.maximum(d0[i, s] + rbuf[i, s], 1.0)
                return inner2
            return lax.fori_loop(0, D // L, _lane, inner)
        lax.fori_loop(0, B, _rcomp, 0)
        pltpu.sync_copy(rbuf, r_out.at[cid, pl.ds(q0, B)])
        return carry
    lax.fori_loop(0, RPT // B, _chunk, 0)
    plsc.subcore_barrier()
    _pipelined_edges(ebase, nb, r_out.at[cid], dst_hbm, src_hbm, c_s,
                     gi, si, rows, gsem, isg, iss)
    plsc.subcore_barrier()
    pltpu.sync_copy(c_s.at[pl.ds(r0, RPT)], c_out.at[cid, pl.ds(r0, RPT)])


@functools.cache
def _sc_calls():
    mesh = plsc.VectorSubcoreMesh(
        core_axis_name="c", subcore_axis_name="s",
        num_cores=NC, num_subcores=NS)
    idx2 = [pltpu.VMEM((B,), jnp.int32)] * (2 * NSLOT)
    rows2 = [pltpu.VMEM((B, D), jnp.float32)] * NSLOT
    sems6 = [pltpu.SemaphoreType.DMA] * (3 * NSLOT)
    edge_agg = pl.kernel(
        _edge_agg,
        out_type=jax.ShapeDtypeStruct((NC, NPAD, D), jnp.float32),
        mesh=mesh,
        scratch_types=(
            [pltpu.VMEM_SHARED((NPAD, D), jnp.float32)]
            + idx2 + rows2 + sems6),
    )
    edge_deg = pl.kernel(
        _edge_deg,
        out_type=jax.ShapeDtypeStruct((NC, NPAD, D), jnp.float32),
        mesh=mesh,
        scratch_types=[
            pltpu.VMEM_SHARED((NPAD, D), jnp.float32),
            pltpu.VMEM((B,), jnp.int32),
            pltpu.VMEM((B,), jnp.int32),
            pltpu.VMEM((B, D), jnp.float32),
            pltpu.SemaphoreType.DMA,
            pltpu.SemaphoreType.DMA,
        ],
    )
    edge_coef = pl.kernel(
        _edge_coef,
        out_type=(jax.ShapeDtypeStruct((NC, NPAD, D), jnp.float32),
                  jax.ShapeDtypeStruct((NC, NPAD, D), jnp.float32)),
        mesh=mesh,
        scratch_types=(
            [pltpu.VMEM_SHARED((NPAD, D), jnp.float32)]
            + idx2 + rows2 + sems6),
    )
    return edge_agg, edge_deg, edge_coef


def _dense_body(h_ref, agg_ref, r_ref, c_ref,
                w1s_ref, w1n_ref, b1_ref, w2s_ref, w2n_ref, b2_ref,
                wc_ref, pf_ref, bc_ref,
                out_ref, s0_ref, s1_ref):
    i = pl.program_id(0)

    @pl.when(i == 0)
    def _():
        s0_ref[...] = jnp.zeros_like(s0_ref)
        s1_ref[...] = jnp.zeros_like(s1_ref)

    agg = agg_ref[0] + agg_ref[1]
    hn = agg * r_ref[:, :1]
    h1 = h_ref[...] @ w1s_ref[...] + hn @ w1n_ref[...] + b1_ref[...]
    h1 = jnp.maximum(h1, 0.0)
    row = i * BR + lax.broadcasted_iota(jnp.int32, (BR, 1), 0)
    valid = (row < N).astype(jnp.float32)
    s0_ref[...] += jnp.sum(h1 * valid, axis=0, keepdims=True)
    cc = (c_ref[0, :, :1] + c_ref[1, :, :1]) * valid
    s1_ref[...] += jnp.sum(h1 * cc, axis=0, keepdims=True)

    @pl.when(i == NPAD // BR - 1)
    def _():
        inv_n = 1.0 / N
        hg = s0_ref[...] * inv_n
        m1 = s1_ref[...] * inv_n
        h2m = hg @ w2s_ref[...] + m1 @ w2n_ref[...] + b2_ref[...]
        out_ref[...] = (h2m @ wc_ref[:D, :] + pf_ref[...] @ wc_ref[D:, :]
                        + bc_ref[...])


_dense_call = pl.pallas_call(
    _dense_body,
    grid=(NPAD // BR,),
    in_specs=[
        pl.BlockSpec((BR, D), lambda i: (i, 0)),
        pl.BlockSpec((NC, BR, D), lambda i: (0, i, 0)),
        pl.BlockSpec((BR, D), lambda i: (i, 0)),
        pl.BlockSpec((NC, BR, D), lambda i: (0, i, 0)),
        pl.BlockSpec((D, HID), lambda i: (0, 0)),
        pl.BlockSpec((D, HID), lambda i: (0, 0)),
        pl.BlockSpec((1, HID), lambda i: (0, 0)),
        pl.BlockSpec((HID, HID), lambda i: (0, 0)),
        pl.BlockSpec((HID, HID), lambda i: (0, 0)),
        pl.BlockSpec((1, HID), lambda i: (0, 0)),
        pl.BlockSpec((HID + PERM, NCLS), lambda i: (0, 0)),
        pl.BlockSpec((1, PERM), lambda i: (0, 0)),
        pl.BlockSpec((1, NCLS), lambda i: (0, 0)),
    ],
    out_specs=pl.BlockSpec((1, NCLS), lambda i: (0, 0)),
    out_shape=jax.ShapeDtypeStruct((1, NCLS), jnp.float32),
    scratch_shapes=[
        pltpu.VMEM((1, HID), jnp.float32),
        pltpu.VMEM((1, HID), jnp.float32),
    ],
)


def kernel(h, edge_index, perm_features, W1_self, W1_neigh, b1,
           W2_self, W2_neigh, b2, Wc, bc):
    edge_agg, edge_deg, edge_coef = _sc_calls()
    ei = edge_index.astype(jnp.int32)
    pad_idx = jnp.full((EPAD + NSLOT * B - E,), N, jnp.int32)
    src = jnp.concatenate([ei[0], pad_idx])
    dst = jnp.concatenate([ei[1], pad_idx])
    h_pad = jnp.pad(h, ((0, NPAD - N), (0, 0)))
    z2 = jnp.zeros((NPAD, D), jnp.float32)

    agg_parts = edge_agg(h_pad, src, dst, z2)
    deg_parts = edge_deg(dst, z2)
    c_parts, r_tab = edge_coef(src, dst, deg_parts, z2)

    return _dense_call(
        h_pad, agg_parts, r_tab[0], c_parts,
        W1_self, W1_neigh, b1.reshape(1, HID),
        W2_self, W2_neigh, b2.reshape(1, HID),
        Wc, perm_features, bc.reshape(1, NCLS))


# speedup vs baseline: 1.0972x; 1.0589x over previous
import functools

import jax
import jax.numpy as jnp
from jax import lax
from jax.experimental import pallas as pl
from jax.experimental.pallas import tpu as pltpu
from jax.experimental.pallas import tpu_sc as plsc

N = 10000
D = 128
HID = 128
NCLS = 10
PERM = 16
NPAD = 10240
E = 320000
NC = 2
NS = 16
L = 16
B = 128
NSLOT = 2
EPW = 10240
EPAD = NC * NS * EPW
NB = EPW // B
EPW0 = 18432
EPW1 = EPW * NC - EPW0
RPT = NPAD // NS
BR = 256


def _worker_slices(cid, sid):
    return sid * RPT, (cid * NS + sid) * EPW


def _gather_split(cid, sid):
    ebase = jnp.where(cid == 0, sid * EPW0, NS * EPW0 + sid * EPW1)
    nb = jnp.where(cid == 0, EPW0 // B, EPW1 // B)
    return ebase, nb


def _pipelined_edges(ebase, nb, tab_hbm, gidx_hbm, sidx_hbm, acc_s,
                     gi, si, rows, gsem, isg, iss):
    def _i_start(b, g):
        b0 = ebase + g * B
        pltpu.async_copy(gidx_hbm.at[pl.ds(b0, B)], gi[b], isg[b])
        pltpu.async_copy(sidx_hbm.at[pl.ds(b0, B)], si[b], iss[b])

    def _i_wait(b):
        pltpu.make_async_copy(gidx_hbm.at[pl.ds(0, B)], gi[b], isg[b]).wait()
        pltpu.make_async_copy(sidx_hbm.at[pl.ds(0, B)], si[b], iss[b]).wait()

    def _g_start(b):
        pltpu.async_copy(tab_hbm.at[gi[b]], rows[b], gsem[b])

    def _g_wait(b):
        pltpu.make_async_copy(tab_hbm.at[gi[b]], rows[b], gsem[b]).wait()

    for r in range(NSLOT - 1):
        _i_start(r, r)
        _i_wait(r)
        _g_start(r)
    _i_start(NSLOT - 1, NSLOT - 1)

    def _body(b, g):
        bn = (b + NSLOT - 1) % NSLOT
        _g_wait(b)
        _i_wait(bn)
        _g_start(bn)
        pltpu.sync_copy(rows[b], acc_s.at[si[b]], add=True)
        _i_start(b, g + NSLOT)

    def _outer(go, carry):
        for u in range(NSLOT):
            _body(u, NSLOT * go + u)
        return carry
    lax.fori_loop(0, nb // NSLOT, _outer, 0)
    for r in range(NSLOT - 1):
        _g_wait(r)
    _i_wait(NSLOT - 1)


def _edge_agg(h_hbm, src_hbm, dst_hbm, z2_hbm, agg_out, agg_s, *bufs):
    cid = lax.axis_index("c")
    sid = lax.axis_index("s")
    r0, _ = _worker_slices(cid, sid)
    ebase, nb = _gather_split(cid, sid)
    pltpu.sync_copy(z2_hbm.at[pl.ds(r0, RPT)], agg_s.at[pl.ds(r0, RPT)])
    plsc.subcore_barrier()
    gi = bufs[0:NSLOT]
    si = bufs[NSLOT:2 * NSLOT]
    rows = bufs[2 * NSLOT:3 * NSLOT]
    gsem = bufs[3 * NSLOT:4 * NSLOT]
    isg = bufs[4 * NSLOT:5 * NSLOT]
    iss = bufs[5 * NSLOT:6 * NSLOT]
    _pipelined_edges(ebase, nb, h_hbm, src_hbm, dst_hbm, agg_s,
                     gi, si, rows, gsem, isg, iss)
    plsc.subcore_barrier()
    pltpu.sync_copy(agg_s.at[pl.ds(r0, RPT)], agg_out.at[cid, pl.ds(r0, RPT)])


def _edge_deg(dst_hbm, zn_hbm,
              deg_out,
              deg_s, di0, di1, ones, isd0, isd1):
    cid = lax.axis_index("c")
    sid = lax.axis_index("s")
    r0, ebase = _worker_slices(cid, sid)
    pltpu.sync_copy(zn_hbm.at[pl.ds(r0, RPT)], deg_s.at[pl.ds(r0, RPT)])

    def _init_ones(i, carry):
        def _lane(k, inner):
            ones[i, pl.ds(k * L, L)] = jnp.ones((L,), jnp.float32)
            return inner
        return lax.fori_loop(0, D // L, _lane, carry)
    lax.fori_loop(0, B, _init_ones, 0)
    plsc.subcore_barrier()

    di = (di0, di1)
    isd = (isd0, isd1)

    def _i_start(b, g):
        pltpu.async_copy(dst_hbm.at[pl.ds(ebase + g * B, B)], di[b], isd[b])

    def _i_wait(b):
        pltpu.make_async_copy(dst_hbm.at[pl.ds(0, B)], di[b], isd[b]).wait()

    _i_start(0, 0)

    def _body(b, g):
        _i_wait(b)
        _i_start(1 - b, g + 1)
        pltpu.sync_copy(ones, deg_s.at[di[b]], add=True)

    def _outer(go, carry):
        _body(0, 2 * go)
        _body(1, 2 * go + 1)
        return carry
    lax.fori_loop(0, NB // 2, _outer, 0)
    _i_wait(0)
    plsc.subcore_barrier()
    pltpu.sync_copy(deg_s.at[pl.ds(r0, RPT)], deg_out.at[cid, pl.ds(r0, RPT)])


def _edge_coef(src_hbm, dst_hbm, degp_hbm, zn_hbm,
               c_out, r_out, c_s, *bufs):
    cid = lax.axis_index("c")
    sid = lax.axis_index("s")
    r0, _ = _worker_slices(cid, sid)
    ebase, nb = _gather_split(cid, sid)
    gi = bufs[0:NSLOT]
    si = bufs[NSLOT:2 * NSLOT]
    rows = bufs[2 * NSLOT:3 * NSLOT]
    gsem = bufs[3 * NSLOT:4 * NSLOT]
    isg = bufs[4 * NSLOT:5 * NSLOT]
    iss = bufs[5 * NSLOT:6 * NSLOT]
    d0, rbuf = rows[0], rows[1]
    pltpu.sync_copy(zn_hbm.at[pl.ds(r0, RPT)], c_s.at[pl.ds(r0, RPT)])
    def _chunk(t, carry):
        q0 = r0 + t * B
        pltpu.sync_copy(degp_hbm.at[0, pl.ds(q0, B)], d0)
        pltpu.sync_copy(degp_hbm.at[1, pl.ds(q0, B)], rbuf)

        def _rcomp(i, inner):
            def _lane(k, inner2):
                s = pl.ds(k * L, L)
                rbuf[i, s] = 1.0 / jnp.maximum(d0[i, s] + rbuf[i, s], 1.0)
                return inner2
            return lax.fori_loop(0, D // L, _lane, inner)
        lax.fori_loop(0, B, _rcomp, 0)
        pltpu.sync_copy(rbuf, r_out.at[cid, pl.ds(q0, B)])
        return carry
    lax.fori_loop(0, RPT // B, _chunk, 0)
    plsc.subcore_barrier()
    _pipelined_edges(ebase, nb, r_out.at[cid], dst_hbm, src_hbm, c_s,
                     gi, si, rows, gsem, isg, iss)
    plsc.subcore_barrier()
    pltpu.sync_copy(c_s.at[pl.ds(r0, RPT)], c_out.at[cid, pl.ds(r0, RPT)])


@functools.cache
def _sc_calls():
    mesh = plsc.VectorSubcoreMesh(
        core_axis_name="c", subcore_axis_name="s",
        num_cores=NC, num_subcores=NS)
    idx2 = [pltpu.VMEM((B,), jnp.int32)] * (2 * NSLOT)
    rows2 = [pltpu.VMEM((B, D), jnp.float32)] * NSLOT
    sems6 = [pltpu.SemaphoreType.DMA] * (3 * NSLOT)
    edge_agg = pl.kernel(
        _edge_agg,
        out_type=jax.ShapeDtypeStruct((NC, NPAD, D), jnp.float32),
        mesh=mesh,
        scratch_types=(
            [pltpu.VMEM_SHARED((NPAD, D), jnp.float32)]
            + idx2 + rows2 + sems6),
    )
    edge_deg = pl.kernel(
        _edge_deg,
        out_type=jax.ShapeDtypeStruct((NC, NPAD, D), jnp.float32),
        mesh=mesh,
        scratch_types=[
            pltpu.VMEM_SHARED((NPAD, D), jnp.float32),
            pltpu.VMEM((B,), jnp.int32),
            pltpu.VMEM((B,), jnp.int32),
            pltpu.VMEM((B, D), jnp.float32),
            pltpu.SemaphoreType.DMA,
            pltpu.SemaphoreType.DMA,
        ],
    )
    edge_coef = pl.kernel(
        _edge_coef,
        out_type=(jax.ShapeDtypeStruct((NC, NPAD, D), jnp.float32),
                  jax.ShapeDtypeStruct((NC, NPAD, D), jnp.float32)),
        mesh=mesh,
        scratch_types=(
            [pltpu.VMEM_SHARED((NPAD, D), jnp.float32)]
            + idx2 + rows2 + sems6),
    )
    return edge_agg, edge_deg, edge_coef


def _dense_body(h_ref, agg_ref, r_ref, c_ref,
                w1s_ref, w1n_ref, b1_ref, w2s_ref, w2n_ref, b2_ref,
                wc_ref, pf_ref, bc_ref,
                out_ref, s0_ref, s1_ref):
    i = pl.program_id(0)

    @pl.when(i == 0)
    def _():
        s0_ref[...] = jnp.zeros_like(s0_ref)
        s1_ref[...] = jnp.zeros_like(s1_ref)

    agg = agg_ref[0] + agg_ref[1]
    hn = agg * r_ref[:, :1]
    h1 = h_ref[...] @ w1s_ref[...] + hn @ w1n_ref[...] + b1_ref[...]
    h1 = jnp.maximum(h1, 0.0)
    row = i * BR + lax.broadcasted_iota(jnp.int32, (BR, 1), 0)
    valid = (row < N).astype(jnp.float32)
    s0_ref[...] += jnp.sum(h1 * valid, axis=0, keepdims=True)
    cc = (c_ref[0, :, :1] + c_ref[1, :, :1]) * valid
    s1_ref[...] += jnp.sum(h1 * cc, axis=0, keepdims=True)

    @pl.when(i == NPAD // BR - 1)
    def _():
        inv_n = 1.0 / N
        hg = s0_ref[...] * inv_n
        m1 = s1_ref[...] * inv_n
        h2m = hg @ w2s_ref[...] + m1 @ w2n_ref[...] + b2_ref[...]
        out_ref[...] = (h2m @ wc_ref[:D, :] + pf_ref[...] @ wc_ref[D:, :]
                        + bc_ref[...])


_dense_call = pl.pallas_call(
    _dense_body,
    grid=(NPAD // BR,),
    in_specs=[
        pl.BlockSpec((BR, D), lambda i: (i, 0)),
        pl.BlockSpec((NC, BR, D), lambda i: (0, i, 0)),
        pl.BlockSpec((BR, D), lambda i: (i, 0)),
        pl.BlockSpec((NC, BR, D), lambda i: (0, i, 0)),
        pl.BlockSpec((D, HID), lambda i: (0, 0)),
        pl.BlockSpec((D, HID), lambda i: (0, 0)),
        pl.BlockSpec((1, HID), lambda i: (0, 0)),
        pl.BlockSpec((HID, HID), lambda i: (0, 0)),
        pl.BlockSpec((HID, HID), lambda i: (0, 0)),
        pl.BlockSpec((1, HID), lambda i: (0, 0)),
        pl.BlockSpec((HID + PERM, NCLS), lambda i: (0, 0)),
        pl.BlockSpec((1, PERM), lambda i: (0, 0)),
        pl.BlockSpec((1, NCLS), lambda i: (0, 0)),
    ],
    out_specs=pl.BlockSpec((1, NCLS), lambda i: (0, 0)),
    out_shape=jax.ShapeDtypeStruct((1, NCLS), jnp.float32),
    scratch_shapes=[
        pltpu.VMEM((1, HID), jnp.float32),
        pltpu.VMEM((1, HID), jnp.float32),
    ],
)


def kernel(h, edge_index, perm_features, W1_self, W1_neigh, b1,
           W2_self, W2_neigh, b2, Wc, bc):
    edge_agg, edge_deg, edge_coef = _sc_calls()
    ei = edge_index.astype(jnp.int32)
    pad_idx = jnp.full((EPAD + NSLOT * B - E,), N, jnp.int32)
    src = jnp.concatenate([ei[0], pad_idx])
    dst = jnp.concatenate([ei[1], pad_idx])
    h_pad = jnp.pad(h, ((0, NPAD - N), (0, 0)))
    z2 = jnp.zeros((NPAD, D), jnp.float32)

    agg_parts = edge_agg(h_pad, src, dst, z2)
    deg_parts = edge_deg(dst, z2)
    c_parts, r_tab = edge_coef(src, dst, deg_parts, z2)

    return _dense_call(
        h_pad, agg_parts, r_tab[0], c_parts,
        W1_self, W1_neigh, b1.reshape(1, HID),
        W2_self, W2_neigh, b2.reshape(1, HID),
        Wc, perm_features, bc.reshape(1, NCLS))
